# Initial kernel scaffold; baseline (speedup 1.0000x reference)
import jax
import jax.numpy as jnp
from jax.experimental import pallas as pl


def kernel(x, w_gate, b_gate):
    raise NotImplementedError("write your pallas kernel here")


# speedup vs baseline: 2.0289x; 2.0289x over previous
import functools

import jax
import jax.numpy as jnp
from jax.experimental import pallas as pl
from jax.experimental.pallas import tpu as pltpu

INPUT_DIM = 2048
NUM_EXPERTS = 64
TOP_K = 8
BATCH = 16384
BLOCK_ROWS = 1024

_NEG_INF = float("-inf")


def _router_kernel(x_ref, w_ref, b_ref, wts_ref, idx_ref, lb_ref, z_ref,
                   f_acc, p_acc, z_acc):
    i = pl.program_id(0)
    n = pl.num_programs(0)

    logits = jax.lax.dot_general(
        x_ref[...], w_ref[...],
        dimension_numbers=(((1,), (1,)), ((), ())),
        preferred_element_type=jnp.float32,
    ) + b_ref[...]

    br = logits.shape[0]
    lane = jax.lax.broadcasted_iota(jnp.int32, (br, NUM_EXPERTS), 1)

    cur = logits
    vals = []
    idxs = []
    for _ in range(TOP_K):
        m = jnp.max(cur, axis=1, keepdims=True)
        hit = cur == m
        ij = jnp.min(jnp.where(hit, lane, NUM_EXPERTS), axis=1,
                     keepdims=True)
        vals.append(m)
        idxs.append(ij)
        cur = jnp.where(lane == ij, _NEG_INF, cur)

    top_vals = jnp.concatenate(vals, axis=1)
    top_idx = jnp.concatenate(idxs, axis=1)

    row_max = vals[0]
    e_top = jnp.exp(top_vals - row_max)
    wts_ref[...] = e_top / jnp.sum(e_top, axis=1, keepdims=True)
    idx_ref[...] = top_idx

    e_all = jnp.exp(logits - row_max)
    denom = jnp.sum(e_all, axis=1, keepdims=True)
    p_part = jnp.sum(e_all / denom, axis=0, keepdims=True)
    lse = row_max + jnp.log(denom)
    z_part = jnp.sum(lse * lse, keepdims=True)

    f_part = jnp.sum(jnp.where(lane == idxs[0], 1.0, 0.0), axis=0,
                     keepdims=True)

    @pl.when(i == 0)
    def _init():
        f_acc[...] = f_part
        p_acc[...] = p_part
        z_acc[...] = z_part

    @pl.when(i != 0)
    def _accum():
        f_acc[...] += f_part
        p_acc[...] += p_part
        z_acc[...] += z_part

    @pl.when(i == n - 1)
    def _finalize():
        inv_b = 1.0 / BATCH
        lb_ref[...] = (float(NUM_EXPERTS) * inv_b * inv_b
                       * jnp.sum(f_acc[...] * p_acc[...], keepdims=True))
        z_ref[...] = z_acc[...] * inv_b


@jax.jit
def kernel(x, w_gate, b_gate):
    b2 = b_gate.reshape(1, NUM_EXPERTS)
    grid = (BATCH // BLOCK_ROWS,)
    wts, idx, lb, z = pl.pallas_call(
        _router_kernel,
        grid=grid,
        in_specs=[
            pl.BlockSpec((BLOCK_ROWS, INPUT_DIM), lambda i: (i, 0)),
            pl.BlockSpec((NUM_EXPERTS, INPUT_DIM), lambda i: (0, 0)),
            pl.BlockSpec((1, NUM_EXPERTS), lambda i: (0, 0)),
        ],
        out_specs=[
            pl.BlockSpec((BLOCK_ROWS, TOP_K), lambda i: (i, 0)),
            pl.BlockSpec((BLOCK_ROWS, TOP_K), lambda i: (i, 0)),
            pl.BlockSpec((1, 1), lambda i: (0, 0)),
            pl.BlockSpec((1, 1), lambda i: (0, 0)),
        ],
        out_shape=[
            jax.ShapeDtypeStruct((BATCH, TOP_K), jnp.float32),
            jax.ShapeDtypeStruct((BATCH, TOP_K), jnp.int32),
            jax.ShapeDtypeStruct((1, 1), jnp.float32),
            jax.ShapeDtypeStruct((1, 1), jnp.float32),
        ],
        scratch_shapes=[
            pltpu.VMEM((1, NUM_EXPERTS), jnp.float32),
            pltpu.VMEM((1, NUM_EXPERTS), jnp.float32),
            pltpu.VMEM((1, 1), jnp.float32),
        ],
    )(x, w_gate, b2)
    return wts, idx, lb[0, 0], z[0, 0]


# speedup vs baseline: 2.2178x; 1.0931x over previous
import functools

import jax
import jax.numpy as jnp
from jax.experimental import pallas as pl
from jax.experimental.pallas import tpu as pltpu

INPUT_DIM = 2048
NUM_EXPERTS = 64
TOP_K = 8
BATCH = 16384
BLOCK_ROWS = 1024

_NEG_INF = float("-inf")


def _router_kernel(x_ref, wt_ref, b_ref, wts_ref, idx_ref, lb_ref, z_ref,
                   f_acc, p_acc, z_acc):
    i = pl.program_id(0)
    n = pl.num_programs(0)

    logits = jax.lax.dot_general(
        x_ref[...], wt_ref[...],
        dimension_numbers=(((1,), (0,)), ((), ())),
        preferred_element_type=jnp.float32,
    ) + b_ref[...]

    br = logits.shape[0]
    lane = jax.lax.broadcasted_iota(jnp.int32, (br, NUM_EXPERTS), 1)

    bits = jax.lax.bitcast_convert_type(logits, jnp.int32)
    sign = jax.lax.shift_right_arithmetic(bits, 31)
    lanecode = lane ^ (jnp.bitwise_not(sign) & 63)
    keys = jax.lax.bitcast_convert_type((bits & -64) | lanecode,
                                        jnp.float32)

    cur = keys
    ms = []
    hit0 = None
    for j in range(TOP_K):
        m = jnp.max(cur, axis=1, keepdims=True)
        ms.append(m)
        hit = cur == m
        if j == 0:
            hit0 = hit
        cur = jnp.where(hit, _NEG_INF, cur)

    mbits = jax.lax.bitcast_convert_type(
        jnp.concatenate(ms, axis=1), jnp.int32)
    msign = jax.lax.shift_right_arithmetic(mbits, 31)
    top_idx = (mbits & 63) ^ (jnp.bitwise_not(msign) & 63)
    top_vals = jax.lax.bitcast_convert_type(mbits & -64, jnp.float32)

    row_max = top_vals[:, :1]
    e_top = jnp.exp(top_vals - row_max)
    wts_ref[...] = e_top / jnp.sum(e_top, axis=1, keepdims=True)
    idx_ref[...] = top_idx

    e_all = jnp.exp(logits - row_max)
    denom = jnp.sum(e_all, axis=1, keepdims=True)
    p_part = jnp.sum(e_all / denom, axis=0, keepdims=True)
    lse = row_max + jnp.log(denom)
    z_part = jnp.sum(lse * lse, keepdims=True)

    f_part = jnp.sum(hit0.astype(jnp.float32), axis=0,
                     keepdims=True)

    @pl.when(i == 0)
    def _init():
        f_acc[...] = f_part
        p_acc[...] = p_part
        z_acc[...] = z_part

    @pl.when(i != 0)
    def _accum():
        f_acc[...] += f_part
        p_acc[...] += p_part
        z_acc[...] += z_part

    @pl.when(i == n - 1)
    def _finalize():
        inv_b = 1.0 / BATCH
        lb_ref[...] = (float(NUM_EXPERTS) * inv_b * inv_b
                       * jnp.sum(f_acc[...] * p_acc[...], keepdims=True))
        z_ref[...] = z_acc[...] * inv_b


@jax.jit
def kernel(x, w_gate, b_gate):
    wt = w_gate.T
    b2 = b_gate.reshape(1, NUM_EXPERTS)
    grid = (BATCH // BLOCK_ROWS,)
    wts, idx, lb, z = pl.pallas_call(
        _router_kernel,
        grid=grid,
        in_specs=[
            pl.BlockSpec((BLOCK_ROWS, INPUT_DIM), lambda i: (i, 0)),
            pl.BlockSpec((INPUT_DIM, NUM_EXPERTS), lambda i: (0, 0)),
            pl.BlockSpec((1, NUM_EXPERTS), lambda i: (0, 0)),
        ],
        out_specs=[
            pl.BlockSpec((BLOCK_ROWS, TOP_K), lambda i: (i, 0)),
            pl.BlockSpec((BLOCK_ROWS, TOP_K), lambda i: (i, 0)),
            pl.BlockSpec((1, 1), lambda i: (0, 0)),
            pl.BlockSpec((1, 1), lambda i: (0, 0)),
        ],
        out_shape=[
            jax.ShapeDtypeStruct((BATCH, TOP_K), jnp.float32),
            jax.ShapeDtypeStruct((BATCH, TOP_K), jnp.int32),
            jax.ShapeDtypeStruct((1, 1), jnp.float32),
            jax.ShapeDtypeStruct((1, 1), jnp.float32),
        ],
        scratch_shapes=[
            pltpu.VMEM((1, NUM_EXPERTS), jnp.float32),
            pltpu.VMEM((1, NUM_EXPERTS), jnp.float32),
            pltpu.VMEM((1, 1), jnp.float32),
        ],
    )(x, wt, b2)
    return wts, idx, lb[0, 0], z[0, 0]
